# baseline (speedup 1.0000x reference)
import numpy as np

import jax
import jax.numpy as jnp
from jax import lax
from jax.experimental import pallas as pl
from jax.experimental.pallas import tpu as pltpu
from jax.experimental.pallas import tpu_sc as plsc

SEQ_LEN = 200
VOCAB = 100000
DIM = 64
BATCH = 4096

NC = 2
NS = 16
LANES = 16
NW = NC * NS
BTILE = BATCH // NW
ST = SEQ_LEN // 8
DT = DIM // 8
GROUPS = DIM // LANES
N_PAIRS = SEQ_LEN // 2

_D = np.arange(DIM)
_FLAT = (_D // 8) * 1024 + (_D % 8) * 128
_TIDX = (np.arange(BTILE)[:, None] + _FLAT[None, :]).astype(np.int32)
_TIDX = _TIDX.reshape(BTILE, GROUPS, LANES)


def _body(idx_hbm, token_hbm, pos_hbm, tidx_hbm, out_hbm,
          idx_v, pos_v, tidx_v, rows_a, rows_b, out_a, out_b,
          gsem_a, gsem_b, osem_a, osem_b):
    c = lax.axis_index("c")
    s_ax = lax.axis_index("s")
    wid = s_ax * NC + c

    pltpu.sync_copy(idx_hbm.at[:, wid, :, :], idx_v)
    pltpu.sync_copy(pos_hbm, pos_v)
    pltpu.sync_copy(tidx_hbm, tidx_v)

    def fire_gather(u, buf, sem):
        pltpu.async_copy(
            token_hbm.at[idx_v.at[u // 8, lax.rem(u, 8), :]], buf, sem)

    def drain(buf, sem):
        pltpu.make_async_copy(token_hbm.at[idx_v.at[0, 0, :]], buf, sem).wait()

    def drain_out(out_t, sem):
        for dt in range(DT):
            pltpu.make_async_copy(out_hbm.at[0, dt, 0, :],
                                  out_t.at[pl.ds(dt * 1024, 1024)], sem).wait()

    def transpose_add(u, rows, out_t):
        pvs = [pos_v[u, pl.ds(g * LANES, LANES)] for g in range(GROUPS)]

        def bb_iter(bb, carry):
            for g in range(GROUPS):
                val = rows[bb, pl.ds(g * LANES, LANES)] + pvs[g]
                plsc.store_scatter(out_t, [tidx_v[bb, g, :]], val)
            return carry
        lax.fori_loop(0, BTILE, bb_iter, 0, unroll=8)

    def writeout(u, out_t, sem):
        for dt in range(DT):
            pltpu.async_copy(out_t.at[pl.ds(dt * 1024, 1024)],
                             out_hbm.at[u, dt, wid, :], sem)

    fire_gather(0, rows_a, gsem_a)
    fire_gather(1, rows_b, gsem_b)

    def pair(t, carry):
        u0 = 2 * t
        u1 = u0 + 1

        drain(rows_a, gsem_a)

        @pl.when(t > 0)
        def _():
            drain_out(out_a, osem_a)
        transpose_add(u0, rows_a, out_a)

        @pl.when(t < N_PAIRS - 1)
        def _():
            fire_gather(u0 + 2, rows_a, gsem_a)
        writeout(u0, out_a, osem_a)

        drain(rows_b, gsem_b)

        @pl.when(t > 0)
        def _():
            drain_out(out_b, osem_b)
        transpose_add(u1, rows_b, out_b)

        @pl.when(t < N_PAIRS - 1)
        def _():
            fire_gather(u1 + 2, rows_b, gsem_b)
        writeout(u1, out_b, osem_b)
        return carry

    lax.fori_loop(0, N_PAIRS, pair, 0)
    drain_out(out_a, osem_a)
    drain_out(out_b, osem_b)


@jax.jit
def _run(idx4, token_table, position_table, tidx):
    mesh = plsc.VectorSubcoreMesh(
        core_axis_name="c", subcore_axis_name="s",
        num_cores=NC, num_subcores=NS,
    )
    fn = pl.kernel(
        _body,
        out_type=jax.ShapeDtypeStruct((SEQ_LEN, DT, NW, 8 * BTILE), jnp.float32),
        mesh=mesh,
        compiler_params=pltpu.CompilerParams(
            use_tc_tiling_on_sc=False, needs_layout_passes=False),
        scratch_types=[
            pltpu.VMEM((ST, 8, BTILE), jnp.int32),
            pltpu.VMEM((SEQ_LEN, DIM), jnp.float32),
            pltpu.VMEM((BTILE, GROUPS, LANES), jnp.int32),
            pltpu.VMEM((BTILE, DIM), jnp.float32),
            pltpu.VMEM((BTILE, DIM), jnp.float32),
            pltpu.VMEM((DT * 8 * BTILE,), jnp.float32),
            pltpu.VMEM((DT * 8 * BTILE,), jnp.float32),
            pltpu.SemaphoreType.DMA,
            pltpu.SemaphoreType.DMA,
            pltpu.SemaphoreType.DMA,
            pltpu.SemaphoreType.DMA,
        ],
    )
    return fn(idx4, token_table, position_table, tidx)


def kernel(inputs, token_table, position_table):
    idx = inputs.astype(jnp.int32)
    idx4 = idx.T.reshape(ST, 8, NW, BTILE).transpose(0, 2, 1, 3)
    out = _run(idx4, token_table, position_table, jnp.asarray(_TIDX))
    out5 = out.reshape(SEQ_LEN, DT, NW, 8, BTILE)
    return out5.transpose(2, 4, 0, 1, 3).reshape(BATCH, SEQ_LEN, DIM)


# speedup vs baseline: 1.4736x; 1.4736x over previous
import numpy as np

import jax
import jax.numpy as jnp
from jax import lax
from jax.experimental import pallas as pl
from jax.experimental.pallas import tpu as pltpu
from jax.experimental.pallas import tpu_sc as plsc

SEQ_LEN = 200
VOCAB = 100000
DIM = 64
BATCH = 4096

NC = 2
NS = 16
LANES = 16
NW = NC * NS
BTILE = BATCH // NW
ST = SEQ_LEN // 8
DT = DIM // 8
GROUPS = DIM // LANES
N_PAIRS = SEQ_LEN // 2

_D = np.arange(DIM)
_FLAT = (_D // 8) * 1024 + (_D % 8) * 128
_TIDX = (np.arange(BTILE)[:, None] + _FLAT[None, :]).astype(np.int32)
_TIDX = _TIDX.reshape(BTILE, GROUPS, LANES)


def _body(idx_hbm, token_hbm, pos_hbm, tidx_hbm, out_hbm,
          idx_v, pos_v, tidx_v, rows_a, rows_b, out_a, out_b,
          gsem_a, gsem_b, osem_a, osem_b):
    c = lax.axis_index("c")
    s_ax = lax.axis_index("s")
    wid = s_ax * NC + c

    pltpu.sync_copy(idx_hbm.at[:, wid, :, :], idx_v)
    pltpu.sync_copy(pos_hbm, pos_v)
    pltpu.sync_copy(tidx_hbm, tidx_v)

    def fire_gather(u, buf, sem):
        pltpu.async_copy(
            token_hbm.at[idx_v.at[u // 8, lax.rem(u, 8), :]], buf, sem)

    def drain(buf, sem):
        pltpu.make_async_copy(token_hbm.at[idx_v.at[0, 0, :]], buf, sem).wait()

    def drain_out(out_t, sem):
        for dt in range(DT):
            pltpu.make_async_copy(out_hbm.at[0, dt, 0, :],
                                  out_t.at[pl.ds(dt * 1024, 1024)], sem).wait()

    def transpose_add(u, rows, out_t):
        pvs = [pos_v[u, pl.ds(g * LANES, LANES)] for g in range(GROUPS)]

        @plsc.parallel_loop(0, BTILE, unroll=8)
        def _(bb):
            for g in range(GROUPS):
                val = rows[bb, pl.ds(g * LANES, LANES)] + pvs[g]
                plsc.store_scatter(out_t, [tidx_v[bb, g, :]], val)

    def writeout(u, out_t, sem):
        for dt in range(DT):
            pltpu.async_copy(out_t.at[pl.ds(dt * 1024, 1024)],
                             out_hbm.at[u, dt, wid, :], sem)

    fire_gather(0, rows_a, gsem_a)
    fire_gather(1, rows_b, gsem_b)

    def pair(t, carry):
        u0 = 2 * t
        u1 = u0 + 1

        drain(rows_a, gsem_a)

        @pl.when(t > 0)
        def _():
            drain_out(out_a, osem_a)
        transpose_add(u0, rows_a, out_a)

        @pl.when(t < N_PAIRS - 1)
        def _():
            fire_gather(u0 + 2, rows_a, gsem_a)
        writeout(u0, out_a, osem_a)

        drain(rows_b, gsem_b)

        @pl.when(t > 0)
        def _():
            drain_out(out_b, osem_b)
        transpose_add(u1, rows_b, out_b)

        @pl.when(t < N_PAIRS - 1)
        def _():
            fire_gather(u1 + 2, rows_b, gsem_b)
        writeout(u1, out_b, osem_b)
        return carry

    lax.fori_loop(0, N_PAIRS, pair, 0)
    drain_out(out_a, osem_a)
    drain_out(out_b, osem_b)


@jax.jit
def _run(idx4, token_table, position_table, tidx):
    mesh = plsc.VectorSubcoreMesh(
        core_axis_name="c", subcore_axis_name="s",
        num_cores=NC, num_subcores=NS,
    )
    fn = pl.kernel(
        _body,
        out_type=jax.ShapeDtypeStruct((SEQ_LEN, DT, NW, 8 * BTILE), jnp.float32),
        mesh=mesh,
        compiler_params=pltpu.CompilerParams(
            use_tc_tiling_on_sc=False, needs_layout_passes=False),
        scratch_types=[
            pltpu.VMEM((ST, 8, BTILE), jnp.int32),
            pltpu.VMEM((SEQ_LEN, DIM), jnp.float32),
            pltpu.VMEM((BTILE, GROUPS, LANES), jnp.int32),
            pltpu.VMEM((BTILE, DIM), jnp.float32),
            pltpu.VMEM((BTILE, DIM), jnp.float32),
            pltpu.VMEM((DT * 8 * BTILE,), jnp.float32),
            pltpu.VMEM((DT * 8 * BTILE,), jnp.float32),
            pltpu.SemaphoreType.DMA,
            pltpu.SemaphoreType.DMA,
            pltpu.SemaphoreType.DMA,
            pltpu.SemaphoreType.DMA,
        ],
    )
    return fn(idx4, token_table, position_table, tidx)


def kernel(inputs, token_table, position_table):
    idx = inputs.astype(jnp.int32)
    idx4 = idx.T.reshape(ST, 8, NW, BTILE).transpose(0, 2, 1, 3)
    out = _run(idx4, token_table, position_table, jnp.asarray(_TIDX))
    out5 = out.reshape(SEQ_LEN, DT, NW, 8, BTILE)
    return out5.transpose(2, 4, 0, 1, 3).reshape(BATCH, SEQ_LEN, DIM)
